# Initial kernel scaffold; baseline (speedup 1.0000x reference)
import jax
import jax.numpy as jnp
from jax.experimental import pallas as pl


def kernel(z, sldj, means, labels):
    raise NotImplementedError("write your pallas kernel here")


# speedup vs baseline: 1.2985x; 1.2985x over previous
import functools
import math

import jax
import jax.numpy as jnp
from jax import lax
from jax.experimental import pallas as pl
from jax.experimental.pallas import tpu as pltpu
from jax.experimental.pallas import tpu_sc as plsc

A = 26
D = 16
C = 100000
B = 16384
TD = A * D

NC = 2
NS = 16
NW = NC * NS
SPW = B // NW
CH = 64
NCHUNK = SPW // CH

_LOG2PI = math.log(2.0 * math.pi)


def _sc_body(z_hbm, sldj_hbm, means_hbm, labels_hbm, out_hbm,
             z_v, lab_v, mu_v, idx_v, sldj_v, out_v, sem):
    wid = lax.axis_index("s") * NC + lax.axis_index("c")

    iota16 = lax.iota(jnp.int32, 16)
    viota26 = iota16 * 26

    def chunk_body(g, carry):
        accq, accs = carry
        base = wid * SPW + g * CH

        pltpu.sync_copy(z_hbm.at[pl.ds(base * TD, CH * TD)], z_v)
        pltpu.sync_copy(labels_hbm.at[pl.ds(base * A, CH * A)], lab_v)
        pltpu.sync_copy(sldj_hbm.at[pl.ds(base, CH)], sldj_v)

        for i in range(A):
            for t in range(CH // 16):
                labs = plsc.load_gather(lab_v, [viota26 + (t * TD + i)])
                idx_v[i, pl.ds(t * 16, 16)] = labs + i * C

        copies = [
            pltpu.async_copy(means_hbm.at[idx_v.at[i]], mu_v.at[i], sem)
            for i in range(A)
        ]
        for c in copies:
            c.wait()

        def sample_body(s, acc):
            zbase = viota26 + s * TD
            for i in range(A):
                vz = plsc.load_gather(z_v, [zbase + i])
                vmu = mu_v[i, s, :]
                dz = vz - vmu
                acc = acc + dz * dz
            return acc

        accq = lax.fori_loop(0, CH, sample_body, accq)

        for t in range(CH // 16):
            accs = accs + sldj_v[pl.ds(t * 16, 16)]
        return accq, accs

    zero = jnp.zeros((16,), jnp.float32)
    accq, accs = lax.fori_loop(0, NCHUNK, chunk_body, (zero, zero))

    out_v[...] = accq * 0.5 - accs
    pltpu.sync_copy(out_v, out_hbm.at[wid])


@jax.jit
def _sc_partials(z_flat, sldj, means_flat, labels_flat):
    mesh = plsc.VectorSubcoreMesh(core_axis_name="c", subcore_axis_name="s")
    run = functools.partial(
        pl.kernel,
        mesh=mesh,
        out_type=jax.ShapeDtypeStruct((NW, 16), jnp.float32),
        compiler_params=pltpu.CompilerParams(
            needs_layout_passes=False, use_tc_tiling_on_sc=False),
        scratch_types=[
            pltpu.VMEM((CH * TD,), jnp.float32),
            pltpu.VMEM((CH * A,), jnp.int32),
            pltpu.VMEM((A, CH, D), jnp.float32),
            pltpu.VMEM((A, CH), jnp.int32),
            pltpu.VMEM((CH,), jnp.float32),
            pltpu.VMEM((16,), jnp.float32),
            pltpu.SemaphoreType.DMA,
        ],
    )(_sc_body)
    return run(z_flat, sldj, means_flat, labels_flat)


def kernel(z, sldj, means, labels):
    z_flat = z.reshape(B * TD)
    means_flat = means.reshape(A * C, D)
    labels_flat = labels.reshape(B * A)
    partials = _sc_partials(z_flat, sldj, means_flat, labels_flat)
    return jnp.sum(partials) / B + 0.5 * TD * _LOG2PI
